# baseline (speedup 1.0000x reference)
import functools

import jax
import jax.numpy as jnp
from jax import lax
from jax.experimental import pallas as pl
from jax.experimental.pallas import tpu as pltpu
from jax.experimental.pallas import tpu_sc as plsc

DIM = 64
BATCH = 16384
LANES = 16
NUM_CORES = 2
NUM_SUBCORES = 16
NUM_WORKERS = NUM_CORES * NUM_SUBCORES
N_ENT = 1000000
WIDE = 2 * DIM

CW = 256
NCHUNKS = (N_ENT + CW - 1) // CW
LAST_C = NCHUNKS - 1
TAIL0 = LAST_C * CW
G_ROWS = NCHUNKS * (CW // 2)
NBODY = 62

BW = BATCH // NUM_WORKERS
CHUNK = 64
NCHUNK = BW // CHUNK
GROUPS = CHUNK // LANES


def _fast_rsqrt(x):
    i = lax.bitcast_convert_type(x, jnp.int32)
    i = jnp.int32(0x5F3759DF) - lax.shift_right_arithmetic(i, 1)
    y = lax.bitcast_convert_type(i, jnp.float32)
    for _ in range(3):
        y = y * (1.5 - 0.5 * x * y * y)
    return y


def _scale(n):
    return jnp.minimum(jnp.float32(1.0), _fast_rsqrt(jnp.maximum(n, jnp.float32(1e-24))))


@functools.cache
def _build_phase1():
    mesh = plsc.VectorSubcoreMesh(
        core_axis_name="c", subcore_axis_name="s",
        num_cores=NUM_CORES, num_subcores=NUM_SUBCORES)

    slab_t = pltpu.VMEM((DIM, CW), jnp.float32)
    obuf_t = pltpu.VMEM((CW // 2, WIDE), jnp.float32)

    @functools.partial(
        pl.kernel,
        out_type=jax.ShapeDtypeStruct((G_ROWS, WIDE), jnp.float32),
        mesh=mesh,
        compiler_params=pltpu.CompilerParams(
            needs_layout_passes=False, use_tc_tiling_on_sc=True),
        scratch_types=[slab_t, slab_t, obuf_t, obuf_t] + [
            pltpu.SemaphoreType.DMA] * 4,
    )
    def phase1(et, tail, g_out,
               slab_a, slab_b, obuf_a, obuf_b, sem_ra, sem_rb,
               sem_wa, sem_wb):
        cid = lax.axis_index("c")
        sid = lax.axis_index("s")
        wid = sid * NUM_CORES + cid
        iota = lax.iota(jnp.int32, LANES)

        def fire_read(c, slab, sem):
            @pl.when(c < LAST_C)
            def _():
                col0 = pl.multiple_of(c * CW, CW)
                pltpu.async_copy(et.at[:, pl.ds(col0, CW)], slab, sem)

            @pl.when(c == LAST_C)
            def _():
                pltpu.async_copy(tail, slab, sem)

        def wait_read(slab, sem):
            pltpu.make_async_copy(et.at[:, pl.ds(0, CW)], slab, sem).wait()

        def fire_write(c, obuf, sem):
            pltpu.async_copy(obuf, g_out.at[pl.ds(c * (CW // 2), CW // 2), :],
                             sem)

        def wait_write(c, obuf, sem):
            pltpu.make_async_copy(
                obuf, g_out.at[pl.ds(c * (CW // 2), CW // 2), :], sem).wait()

        def transpose(slab, obuf):
            @plsc.parallel_loop(0, CW // 2 // LANES, 1)
            def grp_body(g):
                rows = iota + g * LANES
                cols = rows * 2
                for d in range(DIM):
                    dl = lax.bitwise_and(iota + d, jnp.int32(DIM - 1))
                    v0 = plsc.load_gather(slab, [dl, cols])
                    plsc.store_scatter(obuf, [rows, dl], v0)
                    v1 = plsc.load_gather(slab, [dl, cols + 1])
                    plsc.store_scatter(obuf, [rows, dl + DIM], v1)

        fire_read(wid, slab_a, sem_ra)
        fire_read(wid + NUM_WORKERS, slab_b, sem_rb)

        def body(j, _):
            for off, slab, obuf, sem_r, sem_w in (
                    (0, slab_a, obuf_a, sem_ra, sem_wa),
                    (NUM_WORKERS, slab_b, obuf_b, sem_rb, sem_wb)):
                c = wid + j * (2 * NUM_WORKERS) + off
                c_prev = c - 2 * NUM_WORKERS
                c_next = c + 2 * NUM_WORKERS

                @pl.when(c_prev >= 0)
                def _():
                    wait_write(c_prev, obuf, sem_w)

                @pl.when(c < NCHUNKS)
                def _():
                    wait_read(slab, sem_r)
                    transpose(slab, obuf)
                    fire_write(c, obuf, sem_w)
                    fire_read(c_next, slab, sem_r)

            return 0

        lax.fori_loop(0, NBODY, body, 0, unroll=False)

        for off, obuf, sem_w in ((0, obuf_a, sem_wa),
                                 (NUM_WORKERS, obuf_b, sem_wb)):
            c_last = wid + (NBODY - 1) * (2 * NUM_WORKERS) + off

            @pl.when(c_last < NCHUNKS)
            def _():
                wait_write(c_last, obuf, sem_w)

    return phase1


def _gram2(bufs_p, bufs_n, rows, offs_p, offs_n):
    zeros = jnp.zeros((LANES,), jnp.float32)
    iota = lax.iota(jnp.int32, LANES)

    def side(bufs, offs, dl, c):
        h_v, t_v, r_v, w_v = bufs
        oh, ot, orr = offs
        h = plsc.load_gather(h_v, [rows, dl + oh])
        t = plsc.load_gather(t_v, [rows, dl + ot])
        r = plsc.load_gather(r_v, [rows, dl + orr])
        w = plsc.load_gather(w_v, [rows, dl + orr])
        return (c[0] + h * h, c[1] + t * t, c[2] + r * r, c[3] + w * w,
                c[4] + h * w, c[5] + t * w, c[6] + h * r, c[7] + h * t,
                c[8] + r * w, c[9] + r * t)

    def step(d, c):
        dl = lax.bitwise_and(iota + d, jnp.int32(DIM - 1))
        return (side(bufs_p, offs_p, dl, c[0]), side(bufs_n, offs_n, dl, c[1]))

    return lax.fori_loop(0, DIM, step, ((zeros,) * 10, (zeros,) * 10))


def _gram(h_v, t_v, r_v, w_v, rows, oh, ot, orr):
    zeros = jnp.zeros((LANES,), jnp.float32)
    iota = lax.iota(jnp.int32, LANES)

    def step(d, c):
        dl = lax.bitwise_and(iota + d, jnp.int32(DIM - 1))
        h = plsc.load_gather(h_v, [rows, dl + oh])
        t = plsc.load_gather(t_v, [rows, dl + ot])
        r = plsc.load_gather(r_v, [rows, dl + orr])
        w = plsc.load_gather(w_v, [rows, dl + orr])
        return (c[0] + h * h, c[1] + t * t, c[2] + r * r, c[3] + w * w,
                c[4] + h * w, c[5] + t * w, c[6] + h * r, c[7] + h * t,
                c[8] + r * w, c[9] + r * t)

    return lax.fori_loop(0, DIM, step, (zeros,) * 10)


def _y_from_gram(g):
    nh, nt, nr, nw, dhw, dtw, dhr, dht, drw, drt = g
    ah = _scale(nh)
    at = _scale(nt)
    ar = _scale(nr)
    aw = _scale(nw)
    ch = ah
    cr = ar
    ct = -at
    cw = aw * aw * (at * dtw - ah * dhw)
    ss = (ch * ch * nh + cr * cr * nr + ct * ct * nt + cw * cw * nw
          + 2.0 * (ch * cr * dhr + ch * ct * dht + ch * cw * dhw
                   + cr * ct * drt + cr * cw * drw + ct * cw * dtw))
    ss = jnp.maximum(ss, jnp.float32(0.0))
    return ss * _fast_rsqrt(jnp.maximum(ss, jnp.float32(1e-30)))


def _half_offset(idx_ref, rows):
    k = plsc.load_gather(idx_ref, [rows])
    return lax.shift_left(lax.bitwise_and(k, jnp.int32(1)), jnp.int32(6))


@functools.cache
def _build_phase2():
    mesh = plsc.VectorSubcoreMesh(
        core_axis_name="c", subcore_axis_name="s",
        num_cores=NUM_CORES, num_subcores=NUM_SUBCORES)

    idx_t = pltpu.VMEM((BW,), jnp.int32)
    row_t = pltpu.VMEM((CHUNK, WIDE), jnp.float32)

    @functools.partial(
        pl.kernel,
        out_type=jax.ShapeDtypeStruct((NUM_WORKERS, LANES), jnp.float32),
        mesh=mesh,
        compiler_params=pltpu.CompilerParams(
            needs_layout_passes=False, use_tc_tiling_on_sc=True),
        scratch_types=[idx_t] * 12 + [row_t] * 8 + [
            pltpu.VMEM((LANES,), jnp.float32),
            pltpu.SemaphoreType.DMA,
        ],
    )
    def phase2(g_in, r_hbm, w_hbm, hp, rp, tp, hn, rn, tn, out_hbm,
               hp_i, rp_i, tp_i, hn_i, rn_i, tn_i,
               hp_d, rp_d, tp_d, hn_d, rn_d, tn_d,
               hp_v, tp_v, rp_v, wp_v, hn_v, tn_v, rn_v, wn_v,
               acc_v, sem):
        cid = lax.axis_index("c")
        sid = lax.axis_index("s")
        wid = sid * NUM_CORES + cid
        base = wid * BW
        pltpu.sync_copy(hp.at[pl.ds(base, BW)], hp_i)
        pltpu.sync_copy(rp.at[pl.ds(base, BW)], rp_i)
        pltpu.sync_copy(tp.at[pl.ds(base, BW)], tp_i)
        pltpu.sync_copy(hn.at[pl.ds(base, BW)], hn_i)
        pltpu.sync_copy(rn.at[pl.ds(base, BW)], rn_i)
        pltpu.sync_copy(tn.at[pl.ds(base, BW)], tn_i)

        def shift_all(j, _):
            s = pl.ds(j * LANES, LANES)
            for src, dst in ((hp_i, hp_d), (rp_i, rp_d), (tp_i, tp_d),
                             (hn_i, hn_d), (rn_i, rn_d), (tn_i, tn_d)):
                dst[s] = lax.shift_right_logical(src[s], jnp.int32(1))
            return 0

        lax.fori_loop(0, BW // LANES, shift_all, 0)

        iota = lax.iota(jnp.int32, LANES)
        acc = jnp.zeros((LANES,), jnp.float32)
        for c in range(NCHUNK):
            off = c * CHUNK
            cps = [
                pltpu.async_copy(g_in.at[hp_d.at[pl.ds(off, CHUNK)]], hp_v, sem),
                pltpu.async_copy(g_in.at[tp_d.at[pl.ds(off, CHUNK)]], tp_v, sem),
                pltpu.async_copy(r_hbm.at[rp_d.at[pl.ds(off, CHUNK)]], rp_v, sem),
                pltpu.async_copy(w_hbm.at[rp_d.at[pl.ds(off, CHUNK)]], wp_v, sem),
                pltpu.async_copy(g_in.at[hn_d.at[pl.ds(off, CHUNK)]], hn_v, sem),
                pltpu.async_copy(g_in.at[tn_d.at[pl.ds(off, CHUNK)]], tn_v, sem),
                pltpu.async_copy(r_hbm.at[rn_d.at[pl.ds(off, CHUNK)]], rn_v, sem),
                pltpu.async_copy(w_hbm.at[rn_d.at[pl.ds(off, CHUNK)]], wn_v, sem),
            ]
            for cp in cps:
                cp.wait()

            def group(g, acc):
                rows = iota + g * LANES
                abs_rows = rows + off
                oh_p = _half_offset(hp_i, abs_rows)
                ot_p = _half_offset(tp_i, abs_rows)
                or_p = _half_offset(rp_i, abs_rows)
                oh_n = _half_offset(hn_i, abs_rows)
                ot_n = _half_offset(tn_i, abs_rows)
                or_n = _half_offset(rn_i, abs_rows)
                gp, gn = _gram2((hp_v, tp_v, rp_v, wp_v),
                                (hn_v, tn_v, rn_v, wn_v), rows,
                                (oh_p, ot_p, or_p), (oh_n, ot_n, or_n))
                yp = _y_from_gram(gp)
                yn = _y_from_gram(gn)
                return acc + jnp.maximum(yp - yn + jnp.float32(1.0),
                                         jnp.float32(0.0))

            acc = lax.fori_loop(0, GROUPS, group, acc)

        acc_v[...] = acc
        pltpu.sync_copy(acc_v, out_hbm.at[wid])

    return phase2


def kernel(e_table, r_table, wr_table, h_pos, r_pos, t_pos, h_neg, r_neg, t_neg):
    p1 = _build_phase1()
    p2 = _build_phase2()
    tail = jnp.pad(e_table[TAIL0:], ((0, CW - (N_ENT - TAIL0)), (0, 0))).T
    g = p1(e_table.T, tail)
    partials = p2(g, r_table.reshape(-1, WIDE), wr_table.reshape(-1, WIDE),
                  h_pos.astype(jnp.int32), r_pos.astype(jnp.int32),
                  t_pos.astype(jnp.int32), h_neg.astype(jnp.int32),
                  r_neg.astype(jnp.int32), t_neg.astype(jnp.int32))
    return jnp.sum(partials)


# speedup vs baseline: 2.1064x; 2.1064x over previous
import functools

import jax
import jax.numpy as jnp
from jax import lax
from jax.experimental import pallas as pl
from jax.experimental.pallas import tpu as pltpu
from jax.experimental.pallas import tpu_sc as plsc

DIM = 64
BATCH = 16384
LANES = 16
NUM_CORES = 2
NUM_SUBCORES = 16
NUM_WORKERS = NUM_CORES * NUM_SUBCORES
N_ENT = 1000000
WIDE = 2 * DIM

CW = 256
NCHUNKS = (N_ENT + CW - 1) // CW
LAST_C = NCHUNKS - 1
TAIL0 = LAST_C * CW
G_ROWS = NCHUNKS * (CW // 2)
NBODY = 62

BW = BATCH // NUM_WORKERS
CHUNK = 64
NCHUNK = BW // CHUNK
GROUPS = CHUNK // LANES


def _fast_rsqrt(x):
    i = lax.bitcast_convert_type(x, jnp.int32)
    i = jnp.int32(0x5F3759DF) - lax.shift_right_arithmetic(i, 1)
    y = lax.bitcast_convert_type(i, jnp.float32)
    for _ in range(3):
        y = y * (1.5 - 0.5 * x * y * y)
    return y


def _scale(n):
    return jnp.minimum(jnp.float32(1.0), _fast_rsqrt(jnp.maximum(n, jnp.float32(1e-24))))


@functools.cache
def _build_phase1():
    mesh = plsc.VectorSubcoreMesh(
        core_axis_name="c", subcore_axis_name="s",
        num_cores=NUM_CORES, num_subcores=NUM_SUBCORES)

    slab_t = pltpu.VMEM((DIM, CW), jnp.float32)
    obuf_t = pltpu.VMEM((CW // 2, WIDE), jnp.float32)

    @functools.partial(
        pl.kernel,
        out_type=jax.ShapeDtypeStruct((G_ROWS, WIDE), jnp.float32),
        mesh=mesh,
        compiler_params=pltpu.CompilerParams(
            needs_layout_passes=False, use_tc_tiling_on_sc=True),
        scratch_types=[slab_t, slab_t, obuf_t, obuf_t] + [
            pltpu.SemaphoreType.DMA] * 4,
    )
    def phase1(et, tail, g_out,
               slab_a, slab_b, obuf_a, obuf_b, sem_ra, sem_rb,
               sem_wa, sem_wb):
        cid = lax.axis_index("c")
        sid = lax.axis_index("s")
        wid = sid * NUM_CORES + cid
        iota = lax.iota(jnp.int32, LANES)

        def fire_read(c, slab, sem):
            @pl.when(c < LAST_C)
            def _():
                col0 = pl.multiple_of(c * CW, CW)
                pltpu.async_copy(et.at[:, pl.ds(col0, CW)], slab, sem)

            @pl.when(c == LAST_C)
            def _():
                pltpu.async_copy(tail, slab, sem)

        def wait_read(slab, sem):
            pltpu.make_async_copy(et.at[:, pl.ds(0, CW)], slab, sem).wait()

        def fire_write(c, obuf, sem):
            pltpu.async_copy(obuf, g_out.at[pl.ds(c * (CW // 2), CW // 2), :],
                             sem)

        def wait_write(c, obuf, sem):
            pltpu.make_async_copy(
                obuf, g_out.at[pl.ds(c * (CW // 2), CW // 2), :], sem).wait()

        def transpose(slab, obuf):
            def grp_body(g, _):
                rows = iota + g * LANES
                cols = rows * 2
                pend = []
                for d in range(DIM):
                    dl = lax.bitwise_and(iota + d, jnp.int32(DIM - 1))
                    v0 = plsc.load_gather(slab, [dl, cols])
                    v1 = plsc.load_gather(slab, [dl, cols + 1])
                    pend.append((dl, v0, v1))
                    if len(pend) > 2:
                        pdl, p0, p1 = pend.pop(0)
                        plsc.store_scatter(obuf, [rows, pdl], p0)
                        plsc.store_scatter(obuf, [rows, pdl + DIM], p1)
                for pdl, p0, p1 in pend:
                    plsc.store_scatter(obuf, [rows, pdl], p0)
                    plsc.store_scatter(obuf, [rows, pdl + DIM], p1)
                return 0

            lax.fori_loop(0, CW // 2 // LANES, grp_body, 0, unroll=False)

        fire_read(wid, slab_a, sem_ra)
        fire_read(wid + NUM_WORKERS, slab_b, sem_rb)

        def body(j, _):
            for off, slab, obuf, sem_r, sem_w in (
                    (0, slab_a, obuf_a, sem_ra, sem_wa),
                    (NUM_WORKERS, slab_b, obuf_b, sem_rb, sem_wb)):
                c = wid + j * (2 * NUM_WORKERS) + off
                c_prev = c - 2 * NUM_WORKERS
                c_next = c + 2 * NUM_WORKERS

                @pl.when(c_prev >= 0)
                def _():
                    wait_write(c_prev, obuf, sem_w)

                @pl.when(c < NCHUNKS)
                def _():
                    wait_read(slab, sem_r)
                    transpose(slab, obuf)
                    fire_write(c, obuf, sem_w)
                    fire_read(c_next, slab, sem_r)

            return 0

        lax.fori_loop(0, NBODY, body, 0, unroll=False)

        for off, obuf, sem_w in ((0, obuf_a, sem_wa),
                                 (NUM_WORKERS, obuf_b, sem_wb)):
            c_last = wid + (NBODY - 1) * (2 * NUM_WORKERS) + off

            @pl.when(c_last < NCHUNKS)
            def _():
                wait_write(c_last, obuf, sem_w)

    return phase1


def _gram2(bufs_p, bufs_n, rows, offs_p, offs_n):
    zeros = jnp.zeros((LANES,), jnp.float32)
    iota = lax.iota(jnp.int32, LANES)

    def side(bufs, offs, dl, c):
        h_v, t_v, r_v, w_v = bufs
        oh, ot, orr = offs
        h = plsc.load_gather(h_v, [rows, dl + oh])
        t = plsc.load_gather(t_v, [rows, dl + ot])
        r = plsc.load_gather(r_v, [rows, dl + orr])
        w = plsc.load_gather(w_v, [rows, dl + orr])
        return (c[0] + h * h, c[1] + t * t, c[2] + r * r, c[3] + w * w,
                c[4] + h * w, c[5] + t * w, c[6] + h * r, c[7] + h * t,
                c[8] + r * w, c[9] + r * t)

    def step(d, c):
        dl = lax.bitwise_and(iota + d, jnp.int32(DIM - 1))
        return (side(bufs_p, offs_p, dl, c[0]), side(bufs_n, offs_n, dl, c[1]))

    return lax.fori_loop(0, DIM, step, ((zeros,) * 10, (zeros,) * 10))


def _gram(h_v, t_v, r_v, w_v, rows, oh, ot, orr):
    zeros = jnp.zeros((LANES,), jnp.float32)
    iota = lax.iota(jnp.int32, LANES)

    def step(d, c):
        dl = lax.bitwise_and(iota + d, jnp.int32(DIM - 1))
        h = plsc.load_gather(h_v, [rows, dl + oh])
        t = plsc.load_gather(t_v, [rows, dl + ot])
        r = plsc.load_gather(r_v, [rows, dl + orr])
        w = plsc.load_gather(w_v, [rows, dl + orr])
        return (c[0] + h * h, c[1] + t * t, c[2] + r * r, c[3] + w * w,
                c[4] + h * w, c[5] + t * w, c[6] + h * r, c[7] + h * t,
                c[8] + r * w, c[9] + r * t)

    return lax.fori_loop(0, DIM, step, (zeros,) * 10)


def _y_from_gram(g):
    nh, nt, nr, nw, dhw, dtw, dhr, dht, drw, drt = g
    ah = _scale(nh)
    at = _scale(nt)
    ar = _scale(nr)
    aw = _scale(nw)
    ch = ah
    cr = ar
    ct = -at
    cw = aw * aw * (at * dtw - ah * dhw)
    ss = (ch * ch * nh + cr * cr * nr + ct * ct * nt + cw * cw * nw
          + 2.0 * (ch * cr * dhr + ch * ct * dht + ch * cw * dhw
                   + cr * ct * drt + cr * cw * drw + ct * cw * dtw))
    ss = jnp.maximum(ss, jnp.float32(0.0))
    return ss * _fast_rsqrt(jnp.maximum(ss, jnp.float32(1e-30)))


def _half_offset(idx_ref, rows):
    k = plsc.load_gather(idx_ref, [rows])
    return lax.shift_left(lax.bitwise_and(k, jnp.int32(1)), jnp.int32(6))


@functools.cache
def _build_phase2():
    mesh = plsc.VectorSubcoreMesh(
        core_axis_name="c", subcore_axis_name="s",
        num_cores=NUM_CORES, num_subcores=NUM_SUBCORES)

    idx_t = pltpu.VMEM((BW,), jnp.int32)
    row_t = pltpu.VMEM((CHUNK, WIDE), jnp.float32)

    @functools.partial(
        pl.kernel,
        out_type=jax.ShapeDtypeStruct((NUM_WORKERS, LANES), jnp.float32),
        mesh=mesh,
        compiler_params=pltpu.CompilerParams(
            needs_layout_passes=False, use_tc_tiling_on_sc=True),
        scratch_types=[idx_t] * 12 + [row_t] * 8 + [
            pltpu.VMEM((LANES,), jnp.float32),
            pltpu.SemaphoreType.DMA,
        ],
    )
    def phase2(g_in, r_hbm, w_hbm, hp, rp, tp, hn, rn, tn, out_hbm,
               hp_i, rp_i, tp_i, hn_i, rn_i, tn_i,
               hp_d, rp_d, tp_d, hn_d, rn_d, tn_d,
               hp_v, tp_v, rp_v, wp_v, hn_v, tn_v, rn_v, wn_v,
               acc_v, sem):
        cid = lax.axis_index("c")
        sid = lax.axis_index("s")
        wid = sid * NUM_CORES + cid
        base = wid * BW
        pltpu.sync_copy(hp.at[pl.ds(base, BW)], hp_i)
        pltpu.sync_copy(rp.at[pl.ds(base, BW)], rp_i)
        pltpu.sync_copy(tp.at[pl.ds(base, BW)], tp_i)
        pltpu.sync_copy(hn.at[pl.ds(base, BW)], hn_i)
        pltpu.sync_copy(rn.at[pl.ds(base, BW)], rn_i)
        pltpu.sync_copy(tn.at[pl.ds(base, BW)], tn_i)

        def shift_all(j, _):
            s = pl.ds(j * LANES, LANES)
            for src, dst in ((hp_i, hp_d), (rp_i, rp_d), (tp_i, tp_d),
                             (hn_i, hn_d), (rn_i, rn_d), (tn_i, tn_d)):
                dst[s] = lax.shift_right_logical(src[s], jnp.int32(1))
            return 0

        lax.fori_loop(0, BW // LANES, shift_all, 0)

        iota = lax.iota(jnp.int32, LANES)
        acc = jnp.zeros((LANES,), jnp.float32)
        for c in range(NCHUNK):
            off = c * CHUNK
            cps = [
                pltpu.async_copy(g_in.at[hp_d.at[pl.ds(off, CHUNK)]], hp_v, sem),
                pltpu.async_copy(g_in.at[tp_d.at[pl.ds(off, CHUNK)]], tp_v, sem),
                pltpu.async_copy(r_hbm.at[rp_d.at[pl.ds(off, CHUNK)]], rp_v, sem),
                pltpu.async_copy(w_hbm.at[rp_d.at[pl.ds(off, CHUNK)]], wp_v, sem),
                pltpu.async_copy(g_in.at[hn_d.at[pl.ds(off, CHUNK)]], hn_v, sem),
                pltpu.async_copy(g_in.at[tn_d.at[pl.ds(off, CHUNK)]], tn_v, sem),
                pltpu.async_copy(r_hbm.at[rn_d.at[pl.ds(off, CHUNK)]], rn_v, sem),
                pltpu.async_copy(w_hbm.at[rn_d.at[pl.ds(off, CHUNK)]], wn_v, sem),
            ]
            for cp in cps:
                cp.wait()

            def group(g, acc):
                rows = iota + g * LANES
                abs_rows = rows + off
                oh_p = _half_offset(hp_i, abs_rows)
                ot_p = _half_offset(tp_i, abs_rows)
                or_p = _half_offset(rp_i, abs_rows)
                oh_n = _half_offset(hn_i, abs_rows)
                ot_n = _half_offset(tn_i, abs_rows)
                or_n = _half_offset(rn_i, abs_rows)
                gp, gn = _gram2((hp_v, tp_v, rp_v, wp_v),
                                (hn_v, tn_v, rn_v, wn_v), rows,
                                (oh_p, ot_p, or_p), (oh_n, ot_n, or_n))
                yp = _y_from_gram(gp)
                yn = _y_from_gram(gn)
                return acc + jnp.maximum(yp - yn + jnp.float32(1.0),
                                         jnp.float32(0.0))

            acc = lax.fori_loop(0, GROUPS, group, acc)

        acc_v[...] = acc
        pltpu.sync_copy(acc_v, out_hbm.at[wid])

    return phase2


def kernel(e_table, r_table, wr_table, h_pos, r_pos, t_pos, h_neg, r_neg, t_neg):
    p1 = _build_phase1()
    p2 = _build_phase2()
    tail = jnp.pad(e_table[TAIL0:], ((0, CW - (N_ENT - TAIL0)), (0, 0))).T
    g = p1(e_table.T, tail)
    partials = p2(g, r_table.reshape(-1, WIDE), wr_table.reshape(-1, WIDE),
                  h_pos.astype(jnp.int32), r_pos.astype(jnp.int32),
                  t_pos.astype(jnp.int32), h_neg.astype(jnp.int32),
                  r_neg.astype(jnp.int32), t_neg.astype(jnp.int32))
    return jnp.sum(partials)
